# Initial kernel scaffold; baseline (speedup 1.0000x reference)
import jax
import jax.numpy as jnp
from jax.experimental import pallas as pl


def kernel(xyz1, xyz2, color1, color2, params):
    raise NotImplementedError("write your pallas kernel here")


# speedup vs baseline: 1.0068x; 1.0068x over previous
import functools

import jax
import jax.numpy as jnp
from jax.experimental import pallas as pl

LEAK = 0.1
FEAT_NEI = 16
FLOW_NEI = 32
NPOINTS = [128, 32, 8]


def _leaky(x):
    return jnp.where(x >= 0, x, LEAK * x)


def _group_norm(x, gamma, beta, groups=4, eps=1e-5):
    B, N, C = x.shape
    xg = x.reshape(B, N, groups, C // groups)
    mean = xg.mean(axis=(1, 3), keepdims=True)
    var = xg.var(axis=(1, 3), keepdims=True)
    xg = (xg - mean) / jnp.sqrt(var + eps)
    return xg.reshape(B, N, C) * gamma + beta


def _conv_block(x, p):
    y = x @ p['W'].T + p['b']
    y = _group_norm(y, p['gamma'], p['beta'])
    return _leaky(y)


def _linear_leaky(x, p):
    return _leaky(x @ p['W'].T + p['b'])


def _knn(query, ref, k):
    d2 = (jnp.sum(query ** 2, -1, keepdims=True)
          + jnp.sum(ref ** 2, -1)[:, None, :]
          - 2.0 * jnp.einsum('bmd,bnd->bmn', query, ref))
    negd, idx = jax.lax.top_k(-d2, k)
    return idx, -negd


def _gather_points(x, idx):
    return jax.vmap(lambda xi, ii: xi[ii])(x, idx)


def _point_conv_d(pc, feat, npoint, p):
    B, N, _ = pc.shape
    stride = N // npoint
    fps = jnp.arange(npoint, dtype=jnp.int32) * stride
    new_pc = jnp.take(pc, fps, axis=1)
    idx, _ = _knn(new_pc, pc, FEAT_NEI)
    nb_pc = _gather_points(pc, idx)
    nb_feat = _gather_points(feat, idx)
    rel = nb_pc - new_pc[:, :, None, :]
    g = jnp.concatenate([nb_feat, rel], axis=-1)
    g = _linear_leaky(g, p)
    return new_pc, jnp.max(g, axis=2)


def _upsample(dense_pc, sparse_pc, sparse_feat):
    idx, d2 = _knn(dense_pc, sparse_pc, 3)
    w = 1.0 / (d2 + 1e-8)
    w = w / jnp.sum(w, axis=-1, keepdims=True)
    nb = _gather_points(sparse_feat, idx)
    return jnp.sum(nb * w[..., None], axis=2)


def _point_warping(pc1, pc2, flow):
    warped1 = pc1 + flow
    idx, d2 = _knn(pc2, warped1, 3)
    w = 1.0 / (d2 + 1e-8)
    w = w / jnp.sum(w, axis=-1, keepdims=True)
    nb_flow = _gather_points(flow, idx)
    return pc2 - jnp.sum(nb_flow * w[..., None], axis=2)


def _point_conv_flow(pc1, pc2, feat1, feat2, ps):
    idx, _ = _knn(pc1, pc2, FLOW_NEI)
    nb_pc2 = _gather_points(pc2, idx)
    nb_f2 = _gather_points(feat2, idx)
    rel = nb_pc2 - pc1[:, :, None, :]
    f1 = jnp.broadcast_to(feat1[:, :, None, :], nb_f2.shape[:3] + (feat1.shape[-1],))
    g = jnp.concatenate([f1, nb_f2, rel], axis=-1)
    for p in ps:
        g = _linear_leaky(g, p)
    return jnp.max(g, axis=2)


def _flow_mlp_body(x_ref, w0, b0, w1, b1, w2, b2, w3, b3, w4, b4,
                   feat_ref, flow_ref):
    h = x_ref[...]
    h = _leaky(h @ w0[...].T + b0[...])
    h = _leaky(h @ w1[...].T + b1[...])
    h = _leaky(h @ w2[...].T + b2[...])
    h = _leaky(h @ w3[...].T + b3[...])
    feat_ref[...] = h
    flow_ref[...] = h @ w4[...].T + b4[...]


def _flow_mlp_l0(x, ps, interpret=False):
    B, N, C = x.shape
    x2 = x.reshape(B * N, C)
    M = B * N
    TM = 1024
    args = []
    for p in ps:
        args.append(p['W'])
        args.append(p['b'].reshape(1, -1))
    wspecs = [pl.BlockSpec(a.shape, lambda i: (0, 0)) for a in args]
    feat, flow = pl.pallas_call(
        _flow_mlp_body,
        grid=(M // TM,),
        in_specs=[pl.BlockSpec((TM, C), lambda i: (i, 0))] + wspecs,
        out_specs=[pl.BlockSpec((TM, 128), lambda i: (i, 0)),
                   pl.BlockSpec((TM, 3), lambda i: (i, 0))],
        out_shape=[jax.ShapeDtypeStruct((M, 128), jnp.float32),
                   jax.ShapeDtypeStruct((M, 3), jnp.float32)],
        interpret=interpret,
    )(x2, *args)
    return feat.reshape(B, N, 128), flow.reshape(B, N, 3)


def _forward_feature(xyz, color, params):
    pc_l = [xyz]
    f = _conv_block(color, params['init_fc'][0])
    f = _conv_block(f, params['init_fc'][1])
    feat_l = [f]
    for l in range(3):
        fij = feat_l[-1]
        for p in params['feat_ijs'][l]:
            fij = _conv_block(fij, p)
        pc_new, feat_new = _point_conv_d(pc_l[-1], fij, NPOINTS[l], params['subsample'][l])
        pc_l.append(pc_new)
        feat_l.append(feat_new)
    c_feat_l = [None, None, None]
    for l in range(2, -1, -1):
        fji = _upsample(pc_l[l], pc_l[l + 1], feat_l[l + 1])
        fji = _conv_block(fji, params['up_deconv'][l])
        c_feat_l[l] = jnp.concatenate([feat_l[l], fji], axis=-1)
    return c_feat_l, feat_l[:3], pc_l[:3]


def _flownet(xyz1, xyz2, color1, color2, params, interpret=False):
    B = xyz1.shape[0]
    xyz = jnp.concatenate([xyz1, xyz2], axis=0)
    color = jnp.concatenate([color1, color2], axis=0)
    cf, lf, pp = _forward_feature(xyz, color, params)
    cf1 = [c[:B] for c in cf]
    cf2 = [c[B:] for c in cf]
    lf1 = [f[:B] for f in lf]
    pp1 = [p[:B] for p in pp]
    pp2 = [p[B:] for p in pp]

    pc_warped = pp2[2]
    new_feat = lf1[2]
    up_flow = None
    flows = [None, None, None]
    for l in range(2, -1, -1):
        cost = _point_conv_flow(pp1[l], pc_warped, cf1[l], cf2[l], params['cv'][l])
        xs = [new_feat, cost] + ([up_flow] if up_flow is not None else [])
        x = jnp.concatenate(xs, axis=-1)
        if l == 0:
            feat, flow = _flow_mlp_l0(x, params['flow'][l], interpret=interpret)
        else:
            for p in params['flow'][l][:-1]:
                x = _linear_leaky(x, p)
            feat, flow = x, x @ params['flow'][l][-1]['W'].T + params['flow'][l][-1]['b']
        flows[l] = flow
        if l > 0:
            both = jnp.concatenate([flow, feat], axis=-1)
            both_up = _upsample(pp1[l - 1], pp1[l], both)
            up_flow = both_up[..., :3]
            feat_up = both_up[..., 3:]
            pc_warped = _point_warping(pp1[l - 1], pp2[l - 1], up_flow)
            new_feat = jnp.concatenate([lf1[l - 1], feat_up], axis=-1)
    return (flows[0].transpose(0, 2, 1), flows[1].transpose(0, 2, 1),
            flows[2].transpose(0, 2, 1))


def kernel(xyz1, xyz2, color1, color2, params):
    return _flownet(xyz1, xyz2, color1, color2, params)


# speedup vs baseline: 7.0377x; 6.9899x over previous
import jax
import jax.numpy as jnp
from jax.experimental import pallas as pl
from jax.experimental.pallas import tpu as pltpu

LEAK = 0.1
FEAT_NEI = 16
FLOW_NEI = 32
NPOINTS = [128, 32, 8]
INF = float('inf')
_USE_WARP = True
_USE_CV = True
_USE_PCD = True


def _leaky(x):
    return jnp.where(x >= 0, x, LEAK * x)


def _group_norm(x, gamma, beta, groups=4, eps=1e-5):
    B, N, C = x.shape
    xg = x.reshape(B, N, groups, C // groups)
    mean = xg.mean(axis=(1, 3), keepdims=True)
    var = xg.var(axis=(1, 3), keepdims=True)
    xg = (xg - mean) / jnp.sqrt(var + eps)
    return xg.reshape(B, N, C) * gamma + beta


def _conv_block(x, p):
    y = x @ p['W'].T + p['b']
    y = _group_norm(y, p['gamma'], p['beta'])
    return _leaky(y)


def _linear_leaky(x, p):
    return _leaky(x @ p['W'].T + p['b'])


def _knn(query, ref, k):
    d2 = (jnp.sum(query ** 2, -1, keepdims=True)
          + jnp.sum(ref ** 2, -1)[:, None, :]
          - 2.0 * jnp.einsum('bmd,bnd->bmn', query, ref))
    negd, idx = jax.lax.top_k(-d2, k)
    return idx, -negd


def _gather_points(x, idx):
    return jax.vmap(lambda xi, ii: xi[ii])(x, idx)


def _upsample(dense_pc, sparse_pc, sparse_feat):
    idx, d2 = _knn(dense_pc, sparse_pc, 3)
    w = 1.0 / (d2 + 1e-8)
    w = w / jnp.sum(w, axis=-1, keepdims=True)
    nb = _gather_points(sparse_feat, idx)
    return jnp.sum(nb * w[..., None], axis=2)


def _point_warping_small(pc1, pc2, flow):
    warped1 = pc1 + flow
    idx, d2 = _knn(pc2, warped1, 3)
    w = 1.0 / (d2 + 1e-8)
    w = w / jnp.sum(w, axis=-1, keepdims=True)
    nb_flow = _gather_points(flow, idx)
    return pc2 - jnp.sum(nb_flow * w[..., None], axis=2)


def _point_conv_flow(pc1, pc2, feat1, feat2, ps):
    idx, _ = _knn(pc1, pc2, FLOW_NEI)
    nb_pc2 = _gather_points(pc2, idx)
    nb_f2 = _gather_points(feat2, idx)
    rel = nb_pc2 - pc1[:, :, None, :]
    f1 = jnp.broadcast_to(feat1[:, :, None, :], nb_f2.shape[:3] + (feat1.shape[-1],))
    g = jnp.concatenate([f1, nb_f2, rel], axis=-1)
    for p in ps:
        g = _linear_leaky(g, p)
    return jnp.max(g, axis=2)


def _mmd(a, b):
    return jax.lax.dot_general(a, b, (((1,), (0,)), ((), ())),
                               preferred_element_type=jnp.float32)


def _mm(a, b):
    return jax.lax.dot_general(a, b, (((1,), (0,)), ((), ())),
                               preferred_element_type=jnp.float32)


def _mmt(a, b):
    return jax.lax.dot_general(a, b, (((1,), (1,)), ((), ())),
                               preferred_element_type=jnp.float32)


def _costvol_body(qaug_ref, raug_ref, f1_ref, t2_ref, a1t_ref, c1t_ref,
                  b1_ref, w2t_ref, b2_ref, out_ref, s_ref, base_ref):
    qaug = qaug_ref[...]
    raug = raug_ref[0]
    s_ref[...] = _mmd(qaug[:, :3], raug[:3]) + raug[3:4]
    t2 = t2_ref[0]
    base = (_mm(f1_ref[...], a1t_ref[...])
            + 0.5 * _mm(qaug[:, :3], c1t_ref[...]) + b1_ref[...])
    base_ref[...] = base
    out_ref[...] = jnp.full(out_ref.shape, -INF, jnp.float32)

    def step(_, carry):
        s = s_ref[...]
        vmin = jnp.min(s, axis=1, keepdims=True)
        m = s <= vmin
        s_ref[...] = jnp.where(m, INF, s)
        g = _mm(m.astype(jnp.float32), t2)
        z = _leaky(base_ref[...] + g)
        h = _leaky(_mm(z, w2t_ref[...]) + b2_ref[...])
        out_ref[...] = jnp.maximum(out_ref[...], h)
        return carry

    jax.lax.fori_loop(0, FLOW_NEI, step, 0)


def _costvol_l0(pc1, pc2, f1, t2, p1, p2, interpret=False):
    B, N, _ = pc1.shape
    TQ = 256
    QT = N // TQ
    qaug = jnp.concatenate([-2.0 * pc1, jnp.ones((B, N, 1), jnp.float32)],
                           axis=-1).reshape(B * N, 4)
    raug = jnp.concatenate([pc2, jnp.sum(pc2 ** 2, -1, keepdims=True)],
                           axis=-1).transpose(0, 2, 1)
    f1r = f1.reshape(B * N, -1)
    a1t = p1['W'][:, :128].T
    c1t = p1['W'][:, 256:259].T
    b1 = p1['b'].reshape(1, -1)
    w2t = p2['W'].T
    b2 = p2['b'].reshape(1, -1)
    out = pl.pallas_call(
        _costvol_body,
        grid=(B, QT),
        in_specs=[
            pl.BlockSpec((TQ, 4), lambda b, t: (b * QT + t, 0)),
            pl.BlockSpec((1, 4, N), lambda b, t: (b, 0, 0)),
            pl.BlockSpec((TQ, 128), lambda b, t: (b * QT + t, 0)),
            pl.BlockSpec((1, N, 64), lambda b, t: (b, 0, 0)),
            pl.BlockSpec((128, 64), lambda b, t: (0, 0)),
            pl.BlockSpec((3, 64), lambda b, t: (0, 0)),
            pl.BlockSpec((1, 64), lambda b, t: (0, 0)),
            pl.BlockSpec((64, 64), lambda b, t: (0, 0)),
            pl.BlockSpec((1, 64), lambda b, t: (0, 0)),
        ],
        out_specs=pl.BlockSpec((TQ, 64), lambda b, t: (b * QT + t, 0)),
        out_shape=jax.ShapeDtypeStruct((B * N, 64), jnp.float32),
        scratch_shapes=[pltpu.VMEM((TQ, N), jnp.float32),
                        pltpu.VMEM((TQ, 64), jnp.float32)],
        interpret=interpret,
    )(qaug, raug, f1r, t2, a1t, c1t, b1, w2t, b2)
    return out.reshape(B, N, 64)


def _warp_body(qaug_ref, raug_ref, q_ref, ft_ref, out_ref, s_ref, acc_ref):
    qaug = qaug_ref[...]
    raug = raug_ref[0]
    s_ref[...] = _mmd(qaug[:, :3], raug[:3]) + raug[3:4] + qaug[:, 4:5]
    ft = ft_ref[0]
    acc_ref[...] = jnp.zeros(acc_ref.shape, jnp.float32)

    def step(_, carry):
        s = s_ref[...]
        vmin = jnp.min(s, axis=1, keepdims=True)
        m = s <= vmin
        s_ref[...] = jnp.where(m, INF, s)
        w = 1.0 / (vmin + 1e-8)
        acc_ref[...] += w * _mmt(m.astype(jnp.float32), ft)
        return carry

    jax.lax.fori_loop(0, 3, step, 0)
    acc = acc_ref[...]
    den = acc[:, 3:4]
    out_ref[...] = q_ref[...] - acc[:, :3] / den


def _point_warping_l0(pc1, pc2, flow, interpret=False):
    B, N, _ = pc1.shape
    TQ = 256
    QT = N // TQ
    warped = pc1 + flow
    qaug = jnp.concatenate([-2.0 * pc2, jnp.ones((B, N, 1), jnp.float32),
                            jnp.sum(pc2 ** 2, -1, keepdims=True)],
                           axis=-1).reshape(B * N, 5)
    raug = jnp.concatenate([warped, jnp.sum(warped ** 2, -1, keepdims=True),
                            jnp.ones((B, N, 1), jnp.float32)],
                           axis=-1).transpose(0, 2, 1)
    ft = jnp.concatenate([flow, jnp.ones((B, N, 1), jnp.float32)],
                         axis=-1).transpose(0, 2, 1)
    q2 = pc2.reshape(B * N, 3)
    out = pl.pallas_call(
        _warp_body,
        grid=(B, QT),
        in_specs=[
            pl.BlockSpec((TQ, 5), lambda b, t: (b * QT + t, 0)),
            pl.BlockSpec((1, 5, N), lambda b, t: (b, 0, 0)),
            pl.BlockSpec((TQ, 3), lambda b, t: (b * QT + t, 0)),
            pl.BlockSpec((1, 4, N), lambda b, t: (b, 0, 0)),
        ],
        out_specs=pl.BlockSpec((TQ, 3), lambda b, t: (b * QT + t, 0)),
        out_shape=jax.ShapeDtypeStruct((B * N, 3), jnp.float32),
        scratch_shapes=[pltpu.VMEM((TQ, N), jnp.float32),
                        pltpu.VMEM((TQ, 4), jnp.float32)],
        interpret=interpret,
    )(qaug, raug, q2, ft)
    return out.reshape(B, N, 3)


def _pcd_body(qaug_ref, raug_ref, feat_ref, pc_ref, a1t_ref, c1t_ref, b1_ref,
              out_ref, s_ref):
    qaug = qaug_ref[0]
    raug = raug_ref[0]
    s_ref[...] = _mmd(qaug[:, :3], raug[:3]) + raug[3:4]
    t = _mm(feat_ref[0], a1t_ref[...]) + _mm(pc_ref[0], c1t_ref[...])
    base = 0.5 * _mm(qaug[:, :3], c1t_ref[...]) + b1_ref[...]
    out_ref[0] = jnp.full(out_ref.shape[1:], -INF, jnp.float32)

    def step(_, carry):
        s = s_ref[...]
        vmin = jnp.min(s, axis=1, keepdims=True)
        m = s <= vmin
        s_ref[...] = jnp.where(m, INF, s)
        g = _mm(m.astype(jnp.float32), t)
        out_ref[0] = jnp.maximum(out_ref[0], _leaky(base + g))
        return carry

    jax.lax.fori_loop(0, FEAT_NEI, step, 0)


def _point_conv_d_l0(pc, feat, npoint, p, interpret=False):
    B, N, _ = pc.shape
    stride = N // npoint
    fps = jnp.arange(npoint, dtype=jnp.int32) * stride
    new_pc = jnp.take(pc, fps, axis=1)
    qaug = jnp.concatenate([-2.0 * new_pc, jnp.ones((B, npoint, 1), jnp.float32)],
                           axis=-1)
    raug = jnp.concatenate([pc, jnp.sum(pc ** 2, -1, keepdims=True)],
                           axis=-1).transpose(0, 2, 1)
    C = p['W'].shape[0]
    cin = p['W'].shape[1] - 3
    a1t = p['W'][:, :cin].T
    c1t = p['W'][:, cin:].T
    b1 = p['b'].reshape(1, -1)
    out = pl.pallas_call(
        _pcd_body,
        grid=(B,),
        in_specs=[
            pl.BlockSpec((1, npoint, 4), lambda b: (b, 0, 0)),
            pl.BlockSpec((1, 4, N), lambda b: (b, 0, 0)),
            pl.BlockSpec((1, N, cin), lambda b: (b, 0, 0)),
            pl.BlockSpec((1, N, 3), lambda b: (b, 0, 0)),
            pl.BlockSpec((cin, C), lambda b: (0, 0)),
            pl.BlockSpec((3, C), lambda b: (0, 0)),
            pl.BlockSpec((1, C), lambda b: (0, 0)),
        ],
        out_specs=pl.BlockSpec((1, npoint, C), lambda b: (b, 0, 0)),
        out_shape=jax.ShapeDtypeStruct((B, npoint, C), jnp.float32),
        scratch_shapes=[pltpu.VMEM((npoint, N), jnp.float32)],
        interpret=interpret,
    )(qaug, raug, feat, pc, a1t, c1t, b1)
    return new_pc, out


def _point_conv_d_small(pc, feat, npoint, p):
    B, N, _ = pc.shape
    stride = N // npoint
    fps = jnp.arange(npoint, dtype=jnp.int32) * stride
    new_pc = jnp.take(pc, fps, axis=1)
    idx, _ = _knn(new_pc, pc, FEAT_NEI)
    nb_pc = _gather_points(pc, idx)
    nb_feat = _gather_points(feat, idx)
    rel = nb_pc - new_pc[:, :, None, :]
    g = jnp.concatenate([nb_feat, rel], axis=-1)
    g = _linear_leaky(g, p)
    return new_pc, jnp.max(g, axis=2)


def _flow_mlp_body(x_ref, w0, b0, w1, b1, w2, b2, w3, b3, w4, b4,
                   feat_ref, flow_ref):
    h = x_ref[...]
    h = _leaky(h @ w0[...].T + b0[...])
    h = _leaky(h @ w1[...].T + b1[...])
    h = _leaky(h @ w2[...].T + b2[...])
    h = _leaky(h @ w3[...].T + b3[...])
    feat_ref[...] = h
    flow_ref[...] = h @ w4[...].T + b4[...]


def _flow_mlp_l0(x, ps, interpret=False):
    B, N, C = x.shape
    x2 = x.reshape(B * N, C)
    M = B * N
    TM = 1024
    args = []
    for p in ps:
        args.append(p['W'])
        args.append(p['b'].reshape(1, -1))
    wspecs = [pl.BlockSpec(a.shape, lambda i: (0, 0)) for a in args]
    feat, flow = pl.pallas_call(
        _flow_mlp_body,
        grid=(M // TM,),
        in_specs=[pl.BlockSpec((TM, C), lambda i: (i, 0))] + wspecs,
        out_specs=[pl.BlockSpec((TM, 128), lambda i: (i, 0)),
                   pl.BlockSpec((TM, 3), lambda i: (i, 0))],
        out_shape=[jax.ShapeDtypeStruct((M, 128), jnp.float32),
                   jax.ShapeDtypeStruct((M, 3), jnp.float32)],
        interpret=interpret,
    )(x2, *args)
    return feat.reshape(B, N, 128), flow.reshape(B, N, 3)


def _forward_feature(xyz, color, params, interpret=False):
    pc_l = [xyz]
    f = _conv_block(color, params['init_fc'][0])
    f = _conv_block(f, params['init_fc'][1])
    feat_l = [f]
    for l in range(3):
        fij = feat_l[-1]
        for p in params['feat_ijs'][l]:
            fij = _conv_block(fij, p)
        if l == 0 and _USE_PCD:
            pc_new, feat_new = _point_conv_d_l0(pc_l[-1], fij, NPOINTS[l],
                                                params['subsample'][l],
                                                interpret=interpret)
        else:
            pc_new, feat_new = _point_conv_d_small(pc_l[-1], fij, NPOINTS[l],
                                                   params['subsample'][l])
        pc_l.append(pc_new)
        feat_l.append(feat_new)
    c_feat_l = [None, None, None]
    for l in range(2, -1, -1):
        fji = _upsample(pc_l[l], pc_l[l + 1], feat_l[l + 1])
        fji = _conv_block(fji, params['up_deconv'][l])
        c_feat_l[l] = jnp.concatenate([feat_l[l], fji], axis=-1)
    return c_feat_l, feat_l[:3], pc_l[:3]


def _flownet(xyz1, xyz2, color1, color2, params, interpret=False):
    B = xyz1.shape[0]
    xyz = jnp.concatenate([xyz1, xyz2], axis=0)
    color = jnp.concatenate([color1, color2], axis=0)
    cf, lf, pp = _forward_feature(xyz, color, params, interpret=interpret)
    cf1 = [c[:B] for c in cf]
    cf2 = [c[B:] for c in cf]
    lf1 = [f[:B] for f in lf]
    pp1 = [p[:B] for p in pp]
    pp2 = [p[B:] for p in pp]

    pc_warped = pp2[2]
    new_feat = lf1[2]
    up_flow = None
    flows = [None, None, None]
    for l in [2, 1]:
        cost = _point_conv_flow(pp1[l], pc_warped, cf1[l], cf2[l], params['cv'][l])
        xs = [new_feat, cost] + ([up_flow] if up_flow is not None else [])
        x = jnp.concatenate(xs, axis=-1)
        for p in params['flow'][l][:-1]:
            x = _linear_leaky(x, p)
        feat, flow = x, x @ params['flow'][l][-1]['W'].T + params['flow'][l][-1]['b']
        flows[l] = flow
        both = jnp.concatenate([flow, feat], axis=-1)
        both_up = _upsample(pp1[l - 1], pp1[l], both)
        up_flow = both_up[..., :3]
        feat_up = both_up[..., 3:]
        if l == 1 and not _USE_WARP:
            pc_warped = _point_warping_small(pp1[l - 1], pp2[l - 1], up_flow)
        elif l == 1:
            pc_warped = _point_warping_l0(pp1[0], pp2[0], up_flow,
                                          interpret=interpret)
        else:
            pc_warped = _point_warping_small(pp1[l - 1], pp2[l - 1], up_flow)
        new_feat = jnp.concatenate([lf1[l - 1], feat_up], axis=-1)

    if not _USE_CV:
        cost = _point_conv_flow(pp1[0], pc_warped, cf1[0], cf2[0], params['cv'][0])
        x = jnp.concatenate([new_feat, cost, up_flow], axis=-1)
        feat, flow = _flow_mlp_l0(x, params['flow'][0], interpret=interpret)
        flows[0] = flow
        return (flows[0].transpose(0, 2, 1), flows[1].transpose(0, 2, 1),
                flows[2].transpose(0, 2, 1))
    p1, p2 = params['cv'][0]
    b1t = p1['W'][:, 128:256].T
    c1t = p1['W'][:, 256:259].T
    t2 = cf2[0] @ b1t + pc_warped @ c1t
    cost = _costvol_l0(pp1[0], pc_warped, cf1[0], t2, p1, p2,
                       interpret=interpret)
    x = jnp.concatenate([new_feat, cost, up_flow], axis=-1)
    feat, flow = _flow_mlp_l0(x, params['flow'][0], interpret=interpret)
    flows[0] = flow
    return (flows[0].transpose(0, 2, 1), flows[1].transpose(0, 2, 1),
            flows[2].transpose(0, 2, 1))


def kernel(xyz1, xyz2, color1, color2, params):
    return _flownet(xyz1, xyz2, color1, color2, params)
